# Initial kernel scaffold; baseline (speedup 1.0000x reference)
import jax
import jax.numpy as jnp
from jax.experimental import pallas as pl


def kernel(x, block1_edge_index, block2_edge_index, pos_edge_index, neg_edge_index, W1, b1, W2, b2):
    raise NotImplementedError("write your pallas kernel here")


# speedup vs baseline: 3.1028x; 3.1028x over previous
import functools

import jax
import jax.numpy as jnp
from jax import lax
from jax.experimental import pallas as pl
from jax.experimental.pallas import tpu as pltpu
from jax.experimental.pallas import tpu_sc as plsc

N = 10000
E = 320000
P = 50000
D_IN = 128
D_HID = 256
D_OUT = 128

NC = 2
NS = 16
NW = NC * NS
L = 16

R = 10240
JUNK = N
TPW = R // NS

DEG_ROWS = 79
DEG_PWP = DEG_ROWS * 128

AGG_ROWS = 80
AGG_PWP = AGG_ROWS * 128

SCO_ROWS = 26
SCO_PWP = SCO_ROWS * 128

_mesh = plsc.VectorSubcoreMesh(
    core_axis_name="c", subcore_axis_name="s", num_cores=NC, num_subcores=NS)


@functools.partial(
    pl.kernel,
    out_type=jax.ShapeDtypeStruct((NC, 4, R), jnp.float32),
    mesh=_mesh,
    scratch_types=[
        pltpu.VMEM_SHARED((R,), jnp.float32),
        pltpu.VMEM_SHARED((R,), jnp.float32),
        pltpu.VMEM_SHARED((R,), jnp.float32),
        pltpu.VMEM_SHARED((R,), jnp.float32),
        pltpu.VMEM((DEG_ROWS, 128), jnp.int32),
        pltpu.VMEM((DEG_ROWS, 128), jnp.float32),
        pltpu.VMEM((TPW,), jnp.float32),
    ],
)
def _deg_kernel(idx_hbm, ones_hbm, zeros_hbm, out_hbm,
                d0, d1, d2, d3, idx_v, ones_v, buf_v):
    c = lax.axis_index("c")
    s = lax.axis_index("s")
    wid = s * NC + c
    degs = (d0, d1, d2, d3)
    for d in degs:
        pltpu.sync_copy(zeros_hbm.at[pl.ds(s * TPW, TPW)],
                        d.at[pl.ds(s * TPW, TPW)])
    plsc.subcore_barrier()
    pltpu.sync_copy(ones_hbm, ones_v)
    for k, d in enumerate(degs):
        pltpu.sync_copy(idx_hbm.at[k, wid], idx_v)

        def body(j, _, d=d):
            pltpu.sync_copy(ones_v.at[j], d.at[idx_v.at[j]], add=True)
            return 0

        lax.fori_loop(0, DEG_ROWS, body, 0)
    plsc.subcore_barrier()
    for k, d in enumerate(degs):
        pltpu.sync_copy(d.at[pl.ds(s * TPW, TPW)], buf_v)
        pltpu.sync_copy(buf_v, out_hbm.at[c, k, pl.ds(s * TPW, TPW)])


@functools.partial(
    pl.kernel,
    out_type=jax.ShapeDtypeStruct((NC, R, 128), jnp.float32),
    mesh=_mesh,
    scratch_types=[
        pltpu.VMEM_SHARED((R, 128), jnp.float32),
        pltpu.VMEM((AGG_ROWS, 128), jnp.int32),
        pltpu.VMEM((AGG_ROWS, 128), jnp.int32),
        pltpu.VMEM((128, 128), jnp.float32),
        pltpu.SemaphoreType.DMA,
    ],
)
def _agg_kernel(tab_hbm, src_hbm, dst_hbm, zrows_hbm, out_hbm,
                agg_sp, sidx_v, didx_v, rows_v, sem):
    c = lax.axis_index("c")
    s = lax.axis_index("s")
    wid = s * NC + c
    pltpu.sync_copy(zrows_hbm, agg_sp.at[pl.ds(s * TPW, TPW)])
    plsc.subcore_barrier()
    pltpu.sync_copy(src_hbm.at[wid], sidx_v)
    pltpu.sync_copy(dst_hbm.at[wid], didx_v)

    def chunk(j, _):
        pltpu.async_copy(tab_hbm.at[sidx_v.at[j]], rows_v, sem).wait()
        pltpu.sync_copy(rows_v, agg_sp.at[didx_v.at[j]], add=True)
        return 0

    lax.fori_loop(0, AGG_ROWS, chunk, 0)
    plsc.subcore_barrier()

    def wb(j, _):
        pltpu.sync_copy(agg_sp.at[pl.ds(s * TPW + j * 128, 128)], rows_v)
        pltpu.sync_copy(rows_v, out_hbm.at[c, pl.ds(s * TPW + j * 128, 128)])
        return 0

    lax.fori_loop(0, TPW // 128, wb, 0)


@functools.partial(
    pl.kernel,
    out_type=jax.ShapeDtypeStruct((NW, SCO_PWP, L), jnp.float32),
    mesh=_mesh,
    scratch_types=[
        pltpu.VMEM((SCO_ROWS, 128), jnp.int32),
        pltpu.VMEM((SCO_ROWS, 128), jnp.int32),
        pltpu.VMEM((128, 128), jnp.float32),
        pltpu.VMEM((128, 128), jnp.float32),
        pltpu.VMEM((128, L), jnp.float32),
        pltpu.SemaphoreType.DMA,
        pltpu.SemaphoreType.DMA,
    ],
)
def _score_kernel(h2_hbm, u_hbm, v_hbm, out_hbm,
                  uidx_v, vidx_v, hu_v, hv_v, part_v, sem_u, sem_v):
    c = lax.axis_index("c")
    s = lax.axis_index("s")
    wid = s * NC + c
    pltpu.sync_copy(u_hbm.at[wid], uidx_v)
    pltpu.sync_copy(v_hbm.at[wid], vidx_v)

    def chunk(j, _):
        cu = pltpu.async_copy(h2_hbm.at[uidx_v.at[j]], hu_v, sem_u)
        cv = pltpu.async_copy(h2_hbm.at[vidx_v.at[j]], hv_v, sem_v)
        cu.wait()
        cv.wait()

        def edge(e, _):
            acc = jnp.zeros((L,), jnp.float32)
            for k in range(128 // L):
                acc = acc + hu_v[e, pl.ds(k * L, L)] * hv_v[e, pl.ds(k * L, L)]
            part_v[e] = acc
            return 0

        lax.fori_loop(0, 128, edge, 0)
        pltpu.sync_copy(part_v, out_hbm.at[wid, pl.ds(j * 128, 128)])
        return 0

    lax.fori_loop(0, SCO_ROWS, chunk, 0)


def _norm(deg):
    return jnp.where(deg > 0, lax.rsqrt(jnp.maximum(deg, 1e-12)), 0.0)


def _xs_body(x_ref, degp_ref, o_ref):
    deg = degp_ref[0, 0, :] + degp_ref[1, 0, :]
    o_ref[...] = x_ref[...] * _norm(deg)[:, None]


def _tc_xs(x_pad, degp):
    return pl.pallas_call(
        _xs_body,
        out_shape=jax.ShapeDtypeStruct((R, 128), jnp.float32),
    )(x_pad, degp)


_MMB = 1280


def _mm_body(aggp_ref, degp_ref, W1_ref, b1_ref, W2_ref, o_ref):
    p = aggp_ref[0] + aggp_ref[1]
    nd1 = _norm(degp_ref[0, 1, :] + degp_ref[1, 1, :])
    ns2 = _norm(degp_ref[0, 2, :] + degp_ref[1, 2, :])
    h1 = jnp.dot(p * nd1[:, None], W1_ref[...],
                 preferred_element_type=jnp.float32) + b1_ref[...]
    h1 = jnp.maximum(h1, 0.0)
    o_ref[...] = jnp.dot(h1 * ns2[:, None], W2_ref[...],
                         preferred_element_type=jnp.float32)


def _tc_mm(aggp, degp, W1, b1, W2):
    grid = R // _MMB
    return pl.pallas_call(
        _mm_body,
        grid=(grid,),
        in_specs=[
            pl.BlockSpec((NC, _MMB, 128), lambda r: (0, r, 0)),
            pl.BlockSpec((NC, 4, _MMB), lambda r: (0, 0, r)),
            pl.BlockSpec((D_IN, D_HID), lambda r: (0, 0)),
            pl.BlockSpec((1, D_HID), lambda r: (0, 0)),
            pl.BlockSpec((D_HID, D_OUT), lambda r: (0, 0)),
        ],
        out_specs=pl.BlockSpec((_MMB, 128), lambda r: (r, 0)),
        out_shape=jax.ShapeDtypeStruct((R, 128), jnp.float32),
    )(aggp, degp, W1, b1.reshape(1, D_HID), W2)


def _h2_body(aggp_ref, degp_ref, b2_ref, o_ref):
    p = aggp_ref[0] + aggp_ref[1]
    nd2 = _norm(degp_ref[0, 3, :] + degp_ref[1, 3, :])
    o_ref[...] = jnp.maximum(p * nd2[:, None] + b2_ref[...], 0.0)


def _tc_h2(aggp, degp, b2):
    grid = R // _MMB
    return pl.pallas_call(
        _h2_body,
        grid=(grid,),
        in_specs=[
            pl.BlockSpec((NC, _MMB, 128), lambda r: (0, r, 0)),
            pl.BlockSpec((NC, 4, _MMB), lambda r: (0, 0, r)),
            pl.BlockSpec((1, D_OUT), lambda r: (0, 0)),
        ],
        out_specs=pl.BlockSpec((_MMB, 128), lambda r: (r, 0)),
        out_shape=jax.ShapeDtypeStruct((R, 128), jnp.float32),
    )(aggp, degp, b2.reshape(1, D_OUT))


def _red_body(part_ref, o_ref):
    o_ref[...] = jnp.sum(part_ref[...], axis=1, keepdims=True)


def _tc_reduce(part_flat):
    rows = NW * SCO_PWP
    grid = 16
    br = rows // grid
    return pl.pallas_call(
        _red_body,
        grid=(grid,),
        in_specs=[pl.BlockSpec((br, L), lambda r: (r, 0))],
        out_specs=pl.BlockSpec((br, 1), lambda r: (r, 0)),
        out_shape=jax.ShapeDtypeStruct((rows, 1), jnp.float32),
    )(part_flat)


def _pad_deg_idx(idx):
    a = idx.reshape(NW, E // NW)
    pad = jnp.full((NW, DEG_PWP - E // NW), JUNK, jnp.int32)
    return jnp.concatenate([a, pad], axis=1).reshape(NW, DEG_ROWS, 128)


def _pad_agg_idx(idx, fill):
    a = idx.reshape(NW, E // NW)
    pad = jnp.full((NW, AGG_PWP - E // NW), fill, jnp.int32)
    return jnp.concatenate([a, pad], axis=1).reshape(NW, AGG_ROWS, 128)


def _pad_sco_idx(idx):
    a = idx.reshape(NW, (2 * P) // NW)
    pad = jnp.zeros((NW, SCO_PWP - (2 * P) // NW), jnp.int32)
    return jnp.concatenate([a, pad], axis=1).reshape(NW, SCO_ROWS, 128)


def kernel(x, block1_edge_index, block2_edge_index, pos_edge_index,
           neg_edge_index, W1, b1, W2, b2):
    ones = jnp.ones((DEG_ROWS, 128), jnp.float32)
    zeros1 = jnp.zeros((R,), jnp.float32)
    zrows = jnp.zeros((TPW, 128), jnp.float32)

    idx4 = jnp.stack([
        _pad_deg_idx(block1_edge_index[0]),
        _pad_deg_idx(block1_edge_index[1]),
        _pad_deg_idx(block2_edge_index[0]),
        _pad_deg_idx(block2_edge_index[1]),
    ])
    degp = _deg_kernel(idx4, ones, zeros1)

    x_pad = jnp.concatenate(
        [x, jnp.zeros((R - N, D_IN), jnp.float32)], axis=0)
    xs = _tc_xs(x_pad, degp)

    src1 = _pad_agg_idx(block1_edge_index[0], 0)
    dst1 = _pad_agg_idx(block1_edge_index[1], JUNK)
    agg1 = _agg_kernel(xs, src1, dst1, zrows)

    y = _tc_mm(agg1, degp, W1, b1, W2)

    src2 = _pad_agg_idx(block2_edge_index[0], 0)
    dst2 = _pad_agg_idx(block2_edge_index[1], JUNK)
    agg2 = _agg_kernel(y, src2, dst2, zrows)

    h2 = _tc_h2(agg2, degp, b2)

    u = _pad_sco_idx(jnp.concatenate([pos_edge_index[0], neg_edge_index[0]]))
    v = _pad_sco_idx(jnp.concatenate([pos_edge_index[1], neg_edge_index[1]]))
    part = _score_kernel(h2, u, v)

    sums = _tc_reduce(part.reshape(NW * SCO_PWP, L))
    s = sums.reshape(NW, SCO_PWP)[:, : (2 * P) // NW].reshape(2 * P, 1)
    return (s[:P], s[P:])


# speedup vs baseline: 3.2054x; 1.0331x over previous
import functools

import jax
import jax.numpy as jnp
from jax import lax
from jax.experimental import pallas as pl
from jax.experimental.pallas import tpu as pltpu
from jax.experimental.pallas import tpu_sc as plsc

N = 10000
E = 320000
P = 50000
D_IN = 128
D_HID = 256
D_OUT = 128

NC = 2
NS = 16
NW = NC * NS
L = 16

R = 10240
JUNK = N
TPW = R // NS

AGG_ROWS = 80
AGG_PWP = AGG_ROWS * 128

SCO_ROWS = 26
SCO_PWP = SCO_ROWS * 128

_mesh = plsc.VectorSubcoreMesh(
    core_axis_name="c", subcore_axis_name="s", num_cores=NC, num_subcores=NS)


@functools.partial(
    pl.kernel,
    out_type=jax.ShapeDtypeStruct((NC, 4, R), jnp.float32),
    mesh=_mesh,
    scratch_types=[
        pltpu.VMEM_SHARED((R,), jnp.float32),
        pltpu.VMEM_SHARED((R,), jnp.float32),
        pltpu.VMEM_SHARED((R,), jnp.float32),
        pltpu.VMEM_SHARED((R,), jnp.float32),
        pltpu.VMEM((AGG_ROWS, 128), jnp.int32),
        pltpu.VMEM((AGG_ROWS, 128), jnp.float32),
        pltpu.VMEM((TPW,), jnp.float32),
    ],
)
def _deg_kernel(idx_hbm, ones_hbm, zeros_hbm, out_hbm,
                d0, d1, d2, d3, idx_v, ones_v, buf_v):
    c = lax.axis_index("c")
    s = lax.axis_index("s")
    wid = s * NC + c
    degs = (d0, d1, d2, d3)
    for d in degs:
        pltpu.sync_copy(zeros_hbm.at[pl.ds(s * TPW, TPW)],
                        d.at[pl.ds(s * TPW, TPW)])
    plsc.subcore_barrier()
    pltpu.sync_copy(ones_hbm, ones_v)
    for k, d in enumerate(degs):
        pltpu.sync_copy(idx_hbm.at[k, wid], idx_v)

        def body(j, _, d=d):
            pltpu.sync_copy(ones_v.at[j], d.at[idx_v.at[j]], add=True)
            return 0

        lax.fori_loop(0, AGG_ROWS, body, 0)
    plsc.subcore_barrier()
    for k, d in enumerate(degs):
        pltpu.sync_copy(d.at[pl.ds(s * TPW, TPW)], buf_v)
        pltpu.sync_copy(buf_v, out_hbm.at[c, k, pl.ds(s * TPW, TPW)])


@functools.partial(
    pl.kernel,
    out_type=jax.ShapeDtypeStruct((NC, R, 128), jnp.float32),
    mesh=_mesh,
    scratch_types=[
        pltpu.VMEM_SHARED((R, 128), jnp.float32),
        pltpu.VMEM((AGG_ROWS, 128), jnp.int32),
        pltpu.VMEM((2, 128), jnp.int32),
        pltpu.VMEM((128, 128), jnp.float32),
        pltpu.VMEM((128, 128), jnp.float32),
        pltpu.SemaphoreType.DMA,
        pltpu.SemaphoreType.DMA,
        pltpu.SemaphoreType.DMA,
    ],
)
def _agg_kernel(tab_hbm, src_hbm, dst_hbm, zrows_hbm, out_hbm,
                agg_sp, sidx_v, didx2_v, rows0_v, rows1_v, sem0, sem1, semd):
    c = lax.axis_index("c")
    s = lax.axis_index("s")
    wid = s * NC + c
    pltpu.sync_copy(zrows_hbm, agg_sp.at[pl.ds(s * TPW, TPW)])
    plsc.subcore_barrier()
    pltpu.sync_copy(src_hbm.at[wid], sidx_v)

    def pair(g, _):
        j0 = g * 2
        j1 = g * 2 + 1
        cd = pltpu.async_copy(dst_hbm.at[wid, pl.ds(j0, 2)], didx2_v, semd)
        c0 = pltpu.async_copy(tab_hbm.at[sidx_v.at[j0]], rows0_v, sem0)
        c1 = pltpu.async_copy(tab_hbm.at[sidx_v.at[j1]], rows1_v, sem1)
        cd.wait()
        c0.wait()
        pltpu.sync_copy(rows0_v, agg_sp.at[didx2_v.at[0]], add=True)
        c1.wait()
        pltpu.sync_copy(rows1_v, agg_sp.at[didx2_v.at[1]], add=True)
        return 0

    lax.fori_loop(0, AGG_ROWS // 2, pair, 0)
    plsc.subcore_barrier()

    def wb(j, _):
        pltpu.sync_copy(agg_sp.at[pl.ds(s * TPW + j * 128, 128)], rows0_v)
        pltpu.sync_copy(rows0_v, out_hbm.at[c, pl.ds(s * TPW + j * 128, 128)])
        return 0

    lax.fori_loop(0, TPW // 128, wb, 0)


@functools.partial(
    pl.kernel,
    out_type=jax.ShapeDtypeStruct((NW, SCO_PWP, L), jnp.float32),
    mesh=_mesh,
    scratch_types=[
        pltpu.VMEM((SCO_ROWS, 128), jnp.int32),
        pltpu.VMEM((SCO_ROWS, 128), jnp.int32),
        pltpu.VMEM((128, 128), jnp.float32),
        pltpu.VMEM((128, 128), jnp.float32),
        pltpu.VMEM((128, 128), jnp.float32),
        pltpu.VMEM((128, 128), jnp.float32),
        pltpu.VMEM((128, L), jnp.float32),
        pltpu.VMEM((128, L), jnp.float32),
        pltpu.SemaphoreType.DMA,
        pltpu.SemaphoreType.DMA,
        pltpu.SemaphoreType.DMA,
        pltpu.SemaphoreType.DMA,
    ],
)
def _score_kernel(h2_hbm, u_hbm, v_hbm, out_hbm,
                  uidx_v, vidx_v, hu0_v, hv0_v, hu1_v, hv1_v,
                  part0_v, part1_v, su0, sv0, su1, sv1):
    c = lax.axis_index("c")
    s = lax.axis_index("s")
    wid = s * NC + c
    pltpu.sync_copy(u_hbm.at[wid], uidx_v)
    pltpu.sync_copy(v_hbm.at[wid], vidx_v)

    def compute(hu_v, hv_v, part_v):
        def edge8(t, _):
            for q in range(8):
                e = t * 8 + q
                acc = hu_v[e, pl.ds(0, L)] * hv_v[e, pl.ds(0, L)]
                for k in range(1, 128 // L):
                    acc = acc + (hu_v[e, pl.ds(k * L, L)]
                                 * hv_v[e, pl.ds(k * L, L)])
                part_v[e] = acc
            return 0

        lax.fori_loop(0, 16, edge8, 0)

    def pair(g, _):
        j0 = g * 2
        j1 = g * 2 + 1
        cu0 = pltpu.async_copy(h2_hbm.at[uidx_v.at[j0]], hu0_v, su0)
        cv0 = pltpu.async_copy(h2_hbm.at[vidx_v.at[j0]], hv0_v, sv0)
        cu1 = pltpu.async_copy(h2_hbm.at[uidx_v.at[j1]], hu1_v, su1)
        cv1 = pltpu.async_copy(h2_hbm.at[vidx_v.at[j1]], hv1_v, sv1)
        cu0.wait()
        cv0.wait()
        compute(hu0_v, hv0_v, part0_v)
        pltpu.sync_copy(part0_v, out_hbm.at[wid, pl.ds(j0 * 128, 128)])
        cu1.wait()
        cv1.wait()
        compute(hu1_v, hv1_v, part1_v)
        pltpu.sync_copy(part1_v, out_hbm.at[wid, pl.ds(j1 * 128, 128)])
        return 0

    lax.fori_loop(0, SCO_ROWS // 2, pair, 0)


def _norm(deg):
    return jnp.where(deg > 0, lax.rsqrt(jnp.maximum(deg, 1e-12)), 0.0)


def _xs_body(x_ref, degp_ref, o_ref):
    deg = degp_ref[0, 0, :] + degp_ref[1, 0, :]
    o_ref[...] = x_ref[...] * _norm(deg)[:, None]


def _tc_xs(x_pad, degp):
    return pl.pallas_call(
        _xs_body,
        out_shape=jax.ShapeDtypeStruct((R, 128), jnp.float32),
    )(x_pad, degp)


_MMB = 1280


def _mm_body(aggp_ref, degp_ref, W1_ref, b1_ref, W2_ref, o_ref):
    p = aggp_ref[0] + aggp_ref[1]
    nd1 = _norm(degp_ref[0, 1, :] + degp_ref[1, 1, :])
    ns2 = _norm(degp_ref[0, 2, :] + degp_ref[1, 2, :])
    h1 = jnp.dot(p * nd1[:, None], W1_ref[...],
                 preferred_element_type=jnp.float32) + b1_ref[...]
    h1 = jnp.maximum(h1, 0.0)
    o_ref[...] = jnp.dot(h1 * ns2[:, None], W2_ref[...],
                         preferred_element_type=jnp.float32)


def _tc_mm(aggp, degp, W1, b1, W2):
    grid = R // _MMB
    return pl.pallas_call(
        _mm_body,
        grid=(grid,),
        in_specs=[
            pl.BlockSpec((NC, _MMB, 128), lambda r: (0, r, 0)),
            pl.BlockSpec((NC, 4, _MMB), lambda r: (0, 0, r)),
            pl.BlockSpec((D_IN, D_HID), lambda r: (0, 0)),
            pl.BlockSpec((1, D_HID), lambda r: (0, 0)),
            pl.BlockSpec((D_HID, D_OUT), lambda r: (0, 0)),
        ],
        out_specs=pl.BlockSpec((_MMB, 128), lambda r: (r, 0)),
        out_shape=jax.ShapeDtypeStruct((R, 128), jnp.float32),
    )(aggp, degp, W1, b1.reshape(1, D_HID), W2)


def _h2_body(aggp_ref, degp_ref, b2_ref, o_ref):
    p = aggp_ref[0] + aggp_ref[1]
    nd2 = _norm(degp_ref[0, 3, :] + degp_ref[1, 3, :])
    o_ref[...] = jnp.maximum(p * nd2[:, None] + b2_ref[...], 0.0)


def _tc_h2(aggp, degp, b2):
    grid = R // _MMB
    return pl.pallas_call(
        _h2_body,
        grid=(grid,),
        in_specs=[
            pl.BlockSpec((NC, _MMB, 128), lambda r: (0, r, 0)),
            pl.BlockSpec((NC, 4, _MMB), lambda r: (0, 0, r)),
            pl.BlockSpec((1, D_OUT), lambda r: (0, 0)),
        ],
        out_specs=pl.BlockSpec((_MMB, 128), lambda r: (r, 0)),
        out_shape=jax.ShapeDtypeStruct((R, 128), jnp.float32),
    )(aggp, degp, b2.reshape(1, D_OUT))


def _red_body(part_ref, o_ref):
    o_ref[...] = jnp.sum(part_ref[...], axis=1, keepdims=True)


def _tc_reduce(part_flat):
    rows = NW * SCO_PWP
    grid = 16
    br = rows // grid
    return pl.pallas_call(
        _red_body,
        grid=(grid,),
        in_specs=[pl.BlockSpec((br, L), lambda r: (r, 0))],
        out_specs=pl.BlockSpec((br, 1), lambda r: (r, 0)),
        out_shape=jax.ShapeDtypeStruct((rows, 1), jnp.float32),
    )(part_flat)


def _pad_edge_idx(idx):
    a = idx.reshape(NW, E // NW)
    pad = jnp.full((NW, AGG_PWP - E // NW), JUNK, jnp.int32)
    return jnp.concatenate([a, pad], axis=1).reshape(NW, AGG_ROWS, 128)


def _pad_sco_idx(idx):
    a = idx.reshape(NW, (2 * P) // NW)
    pad = jnp.zeros((NW, SCO_PWP - (2 * P) // NW), jnp.int32)
    return jnp.concatenate([a, pad], axis=1).reshape(NW, SCO_ROWS, 128)


def kernel(x, block1_edge_index, block2_edge_index, pos_edge_index,
           neg_edge_index, W1, b1, W2, b2):
    ones = jnp.ones((AGG_ROWS, 128), jnp.float32)
    zeros1 = jnp.zeros((R,), jnp.float32)
    zrows = jnp.zeros((TPW, 128), jnp.float32)

    src1 = _pad_edge_idx(block1_edge_index[0])
    dst1 = _pad_edge_idx(block1_edge_index[1])
    src2 = _pad_edge_idx(block2_edge_index[0])
    dst2 = _pad_edge_idx(block2_edge_index[1])

    idx4 = jnp.stack([src1, dst1, src2, dst2])
    degp = _deg_kernel(idx4, ones, zeros1)

    x_pad = jnp.concatenate(
        [x, jnp.zeros((R - N, D_IN), jnp.float32)], axis=0)
    xs = _tc_xs(x_pad, degp)

    agg1 = _agg_kernel(xs, src1, dst1, zrows)
    y = _tc_mm(agg1, degp, W1, b1, W2)
    agg2 = _agg_kernel(y, src2, dst2, zrows)
    h2 = _tc_h2(agg2, degp, b2)

    u = _pad_sco_idx(jnp.concatenate([pos_edge_index[0], neg_edge_index[0]]))
    v = _pad_sco_idx(jnp.concatenate([pos_edge_index[1], neg_edge_index[1]]))
    part = _score_kernel(h2, u, v)

    sums = _tc_reduce(part.reshape(NW * SCO_PWP, L))
    s = sums.reshape(NW, SCO_PWP)[:, : (2 * P) // NW].reshape(2 * P, 1)
    return (s[:P], s[P:])


# speedup vs baseline: 4.7679x; 1.4875x over previous
import functools

import jax
import jax.numpy as jnp
from jax import lax
from jax.experimental import pallas as pl
from jax.experimental.pallas import tpu as pltpu
from jax.experimental.pallas import tpu_sc as plsc

N = 10000
E = 320000
P = 50000
D_IN = 128
D_HID = 256
D_OUT = 128

NC = 2
NS = 16
NW = NC * NS
L = 16

R = 10240
JUNK = N
TPW = R // NS

AGG_CH = 64
AGG_NCH = 160
AGG_PWP = AGG_NCH * AGG_CH

SCO_CH = 64
SCO_NCH = 52
SCO_PWP = SCO_NCH * SCO_CH
SCO_IR = SCO_NCH // 2

DEG_ROWS = 80

_mesh = plsc.VectorSubcoreMesh(
    core_axis_name="c", subcore_axis_name="s", num_cores=NC, num_subcores=NS)


@functools.partial(
    pl.kernel,
    out_type=jax.ShapeDtypeStruct((NC, 4, R), jnp.float32),
    mesh=_mesh,
    scratch_types=[
        pltpu.VMEM_SHARED((R,), jnp.float32),
        pltpu.VMEM_SHARED((R,), jnp.float32),
        pltpu.VMEM_SHARED((R,), jnp.float32),
        pltpu.VMEM_SHARED((R,), jnp.float32),
        pltpu.VMEM((DEG_ROWS, 128), jnp.int32),
        pltpu.VMEM((DEG_ROWS, 128), jnp.float32),
    ],
)
def _deg_kernel(idx_hbm, ones_hbm, zeros_hbm, out_hbm,
                d0, d1, d2, d3, idx_v, ones_v):
    c = lax.axis_index("c")
    s = lax.axis_index("s")
    wid = s * NC + c
    degs = (d0, d1, d2, d3)
    for d in degs:
        pltpu.sync_copy(zeros_hbm.at[pl.ds(s * TPW, TPW)],
                        d.at[pl.ds(s * TPW, TPW)])
    plsc.subcore_barrier()
    pltpu.sync_copy(ones_hbm, ones_v)
    for k, d in enumerate(degs):
        pltpu.sync_copy(idx_hbm.at[k, wid], idx_v)

        def body(j, _, d=d):
            pltpu.sync_copy(ones_v.at[j], d.at[idx_v.at[j]], add=True)
            return 0

        lax.fori_loop(0, DEG_ROWS, body, 0)
    plsc.subcore_barrier()
    for k, d in enumerate(degs):
        pltpu.sync_copy(d.at[pl.ds(s * TPW, TPW)],
                        out_hbm.at[c, k, pl.ds(s * TPW, TPW)])


@functools.partial(
    pl.kernel,
    out_type=jax.ShapeDtypeStruct((NC, R, 128), jnp.float32),
    mesh=_mesh,
    scratch_types=[
        pltpu.VMEM_SHARED((R, 128), jnp.float32),
        pltpu.VMEM((8, AGG_CH), jnp.int32),
        pltpu.VMEM((8, AGG_CH), jnp.int32),
        pltpu.VMEM((AGG_CH, 128), jnp.float32),
        pltpu.VMEM((AGG_CH, 128), jnp.float32),
        pltpu.VMEM((AGG_CH, 128), jnp.float32),
        pltpu.VMEM((AGG_CH, 128), jnp.float32),
        pltpu.SemaphoreType.DMA,
        pltpu.SemaphoreType.DMA,
        pltpu.SemaphoreType.DMA,
        pltpu.SemaphoreType.DMA,
        pltpu.SemaphoreType.DMA,
        pltpu.SemaphoreType.DMA,
    ],
)
def _agg_kernel(tab_hbm, src_hbm, dst_hbm, zrows_hbm, out_hbm,
                agg_sp, sidx_v, didx_v, r0_v, r1_v, r2_v, r3_v,
                sg0, sg1, sg2, sg3, semd, sems):
    c = lax.axis_index("c")
    s = lax.axis_index("s")
    wid = s * NC + c
    pltpu.sync_copy(zrows_hbm, agg_sp.at[pl.ds(s * TPW, TPW)])
    plsc.subcore_barrier()

    rows = (r0_v, r1_v, r2_v, r3_v)
    gsems = (sg0, sg1, sg2, sg3)

    n_iter = AGG_NCH // 4
    pltpu.async_copy(src_hbm.at[wid, pl.ds(0, 4)],
                     sidx_v.at[pl.ds(0, 4)], semd)
    pltpu.async_copy(dst_hbm.at[wid, pl.ds(0, 4)],
                     didx_v.at[pl.ds(0, 4)], semd)

    def ring(g, _):
        p4 = (g % 2) * 4
        pltpu.make_async_copy(src_hbm.at[wid, pl.ds(0, 4)],
                              sidx_v.at[pl.ds(0, 4)], semd).wait()
        pltpu.make_async_copy(dst_hbm.at[wid, pl.ds(0, 4)],
                              didx_v.at[pl.ds(0, 4)], semd).wait()
        gc = []
        for b in range(4):
            @pl.when(g > 0)
            def _(b=b):
                pltpu.make_async_copy(
                    rows[b], agg_sp.at[didx_v.at[0]], sems).wait()

            gc.append(pltpu.async_copy(
                tab_hbm.at[sidx_v.at[p4 + b]], rows[b], gsems[b]))
        gn = lax.min(g + 1, n_iter - 1)
        pn = ((g + 1) % 2) * 4
        pltpu.async_copy(src_hbm.at[wid, pl.ds(gn * 4, 4)],
                         sidx_v.at[pl.ds(pn, 4)], semd)
        pltpu.async_copy(dst_hbm.at[wid, pl.ds(gn * 4, 4)],
                         didx_v.at[pl.ds(pn, 4)], semd)
        for b in range(4):
            gc[b].wait()
            pltpu.async_copy(rows[b], agg_sp.at[didx_v.at[p4 + b]],
                             sems, add=True)
        return 0

    lax.fori_loop(0, n_iter, ring, 0)
    pltpu.make_async_copy(src_hbm.at[wid, pl.ds(0, 4)],
                          sidx_v.at[pl.ds(0, 4)], semd).wait()
    pltpu.make_async_copy(dst_hbm.at[wid, pl.ds(0, 4)],
                          didx_v.at[pl.ds(0, 4)], semd).wait()
    for b in range(4):
        pltpu.make_async_copy(rows[b], agg_sp.at[didx_v.at[0]], sems).wait()
    plsc.subcore_barrier()
    pltpu.sync_copy(agg_sp.at[pl.ds(s * TPW, TPW)],
                    out_hbm.at[c, pl.ds(s * TPW, TPW)])


@functools.partial(
    pl.kernel,
    out_type=jax.ShapeDtypeStruct((NW, SCO_PWP * L), jnp.float32),
    mesh=_mesh,
    scratch_types=[
        pltpu.VMEM_SHARED((R, 128), jnp.float32),
        pltpu.VMEM((SCO_IR, 128), jnp.int32),
        pltpu.VMEM((SCO_IR, 128), jnp.int32),
        pltpu.VMEM((SCO_CH, 128), jnp.float32),
        pltpu.VMEM((SCO_CH, 128), jnp.float32),
        pltpu.VMEM((SCO_CH, 128), jnp.float32),
        pltpu.VMEM((SCO_CH, 128), jnp.float32),
        pltpu.VMEM((SCO_CH * L,), jnp.float32),
        pltpu.VMEM((SCO_CH * L,), jnp.float32),
        pltpu.SemaphoreType.DMA,
        pltpu.SemaphoreType.DMA,
        pltpu.SemaphoreType.DMA,
        pltpu.SemaphoreType.DMA,
        pltpu.SemaphoreType.DMA,
    ],
)
def _score_kernel(h2_hbm, u_hbm, v_hbm, out_hbm,
                  tab_sp, uidx_v, vidx_v, hu0_v, hv0_v, hu1_v, hv1_v,
                  part0_v, part1_v, su0, sv0, su1, sv1, swb):
    c = lax.axis_index("c")
    s = lax.axis_index("s")
    wid = s * NC + c
    pltpu.sync_copy(h2_hbm.at[pl.ds(s * TPW, TPW)],
                    tab_sp.at[pl.ds(s * TPW, TPW)])
    pltpu.sync_copy(u_hbm.at[wid], uidx_v)
    pltpu.sync_copy(v_hbm.at[wid], vidx_v)
    plsc.subcore_barrier()

    PB = SCO_CH * L

    def compute(hu_v, hv_v, part_v):
        def edge8(t, _):
            for q in range(8):
                e = t * 8 + q
                acc = hu_v[e, pl.ds(0, L)] * hv_v[e, pl.ds(0, L)]
                for k in range(1, 128 // L):
                    acc = acc + (hu_v[e, pl.ds(k * L, L)]
                                 * hv_v[e, pl.ds(k * L, L)])
                part_v[pl.ds(e * L, L)] = acc
            return 0

        lax.fori_loop(0, SCO_CH // 8, edge8, 0)

    def pair(g, _):
        cu0 = pltpu.async_copy(
            tab_sp.at[uidx_v.at[g, pl.ds(0, SCO_CH)]], hu0_v, su0)
        cv0 = pltpu.async_copy(
            tab_sp.at[vidx_v.at[g, pl.ds(0, SCO_CH)]], hv0_v, sv0)
        cu1 = pltpu.async_copy(
            tab_sp.at[uidx_v.at[g, pl.ds(SCO_CH, SCO_CH)]], hu1_v, su1)
        cv1 = pltpu.async_copy(
            tab_sp.at[vidx_v.at[g, pl.ds(SCO_CH, SCO_CH)]], hv1_v, sv1)
        cu0.wait()
        cv0.wait()

        @pl.when(g > 0)
        def _():
            pltpu.make_async_copy(
                part0_v, out_hbm.at[wid, pl.ds(0, PB)], swb).wait()

        compute(hu0_v, hv0_v, part0_v)
        pltpu.async_copy(part0_v, out_hbm.at[wid, pl.ds(g * 2 * PB, PB)], swb)
        cu1.wait()
        cv1.wait()

        @pl.when(g > 0)
        def _():
            pltpu.make_async_copy(
                part1_v, out_hbm.at[wid, pl.ds(0, PB)], swb).wait()

        compute(hu1_v, hv1_v, part1_v)
        pltpu.async_copy(part1_v,
                         out_hbm.at[wid, pl.ds((g * 2 + 1) * PB, PB)], swb)
        return 0

    lax.fori_loop(0, SCO_IR, pair, 0)
    pltpu.make_async_copy(part0_v, out_hbm.at[wid, pl.ds(0, PB)], swb).wait()
    pltpu.make_async_copy(part1_v, out_hbm.at[wid, pl.ds(0, PB)], swb).wait()


def _norm(deg):
    return jnp.where(deg > 0, lax.rsqrt(jnp.maximum(deg, 1e-12)), 0.0)


def _xs_body(x_ref, degp_ref, o_ref):
    deg = degp_ref[0, 0, :] + degp_ref[1, 0, :]
    o_ref[...] = x_ref[...] * _norm(deg)[:, None]


def _tc_xs(x_pad, degp):
    return pl.pallas_call(
        _xs_body,
        out_shape=jax.ShapeDtypeStruct((R, 128), jnp.float32),
    )(x_pad, degp)


_MMB = 1280


def _mm_body(aggp_ref, degp_ref, W1_ref, b1_ref, W2_ref, o_ref):
    p = aggp_ref[0] + aggp_ref[1]
    nd1 = _norm(degp_ref[0, 1, :] + degp_ref[1, 1, :])
    ns2 = _norm(degp_ref[0, 2, :] + degp_ref[1, 2, :])
    h1 = jnp.dot(p * nd1[:, None], W1_ref[...],
                 preferred_element_type=jnp.float32) + b1_ref[...]
    h1 = jnp.maximum(h1, 0.0)
    o_ref[...] = jnp.dot(h1 * ns2[:, None], W2_ref[...],
                         preferred_element_type=jnp.float32)


def _tc_mm(aggp, degp, W1, b1, W2):
    grid = R // _MMB
    return pl.pallas_call(
        _mm_body,
        grid=(grid,),
        in_specs=[
            pl.BlockSpec((NC, _MMB, 128), lambda r: (0, r, 0)),
            pl.BlockSpec((NC, 4, _MMB), lambda r: (0, 0, r)),
            pl.BlockSpec((D_IN, D_HID), lambda r: (0, 0)),
            pl.BlockSpec((1, D_HID), lambda r: (0, 0)),
            pl.BlockSpec((D_HID, D_OUT), lambda r: (0, 0)),
        ],
        out_specs=pl.BlockSpec((_MMB, 128), lambda r: (r, 0)),
        out_shape=jax.ShapeDtypeStruct((R, 128), jnp.float32),
    )(aggp, degp, W1, b1.reshape(1, D_HID), W2)


def _h2_body(aggp_ref, degp_ref, b2_ref, o_ref):
    p = aggp_ref[0] + aggp_ref[1]
    nd2 = _norm(degp_ref[0, 3, :] + degp_ref[1, 3, :])
    o_ref[...] = jnp.maximum(p * nd2[:, None] + b2_ref[...], 0.0)


def _tc_h2(aggp, degp, b2):
    grid = R // _MMB
    return pl.pallas_call(
        _h2_body,
        grid=(grid,),
        in_specs=[
            pl.BlockSpec((NC, _MMB, 128), lambda r: (0, r, 0)),
            pl.BlockSpec((NC, 4, _MMB), lambda r: (0, 0, r)),
            pl.BlockSpec((1, D_OUT), lambda r: (0, 0)),
        ],
        out_specs=pl.BlockSpec((_MMB, 128), lambda r: (r, 0)),
        out_shape=jax.ShapeDtypeStruct((R, 128), jnp.float32),
    )(aggp, degp, b2.reshape(1, D_OUT))


def _red_body(part_ref, o_ref):
    o_ref[...] = jnp.sum(part_ref[...], axis=1, keepdims=True)


def _tc_reduce(part_flat):
    rows = NW * SCO_PWP
    grid = 16
    br = rows // grid
    return pl.pallas_call(
        _red_body,
        grid=(grid,),
        in_specs=[pl.BlockSpec((br, L), lambda r: (r, 0))],
        out_specs=pl.BlockSpec((br, 1), lambda r: (r, 0)),
        out_shape=jax.ShapeDtypeStruct((rows, 1), jnp.float32),
    )(part_flat)


def _pad_edge_idx(idx):
    a = idx.reshape(NW, E // NW)
    pad = jnp.full((NW, AGG_PWP - E // NW), JUNK, jnp.int32)
    return jnp.concatenate([a, pad], axis=1)


def _pad_sco_idx(idx):
    a = idx.reshape(NW, (2 * P) // NW)
    pad = jnp.zeros((NW, SCO_PWP - (2 * P) // NW), jnp.int32)
    return jnp.concatenate([a, pad], axis=1).reshape(NW, SCO_IR, 128)


def kernel(x, block1_edge_index, block2_edge_index, pos_edge_index,
           neg_edge_index, W1, b1, W2, b2):
    ones = jnp.ones((DEG_ROWS, 128), jnp.float32)
    zeros1 = jnp.zeros((R,), jnp.float32)
    zrows = jnp.zeros((TPW, 128), jnp.float32)

    src1 = _pad_edge_idx(block1_edge_index[0])
    dst1 = _pad_edge_idx(block1_edge_index[1])
    src2 = _pad_edge_idx(block2_edge_index[0])
    dst2 = _pad_edge_idx(block2_edge_index[1])

    idx4 = jnp.stack([src1, dst1, src2, dst2]).reshape(4, NW, DEG_ROWS, 128)
    degp = _deg_kernel(idx4, ones, zeros1)

    x_pad = jnp.concatenate(
        [x, jnp.zeros((R - N, D_IN), jnp.float32)], axis=0)
    xs = _tc_xs(x_pad, degp)

    a_shape = (NW, AGG_NCH, AGG_CH)
    agg1 = _agg_kernel(xs, src1.reshape(a_shape), dst1.reshape(a_shape),
                       zrows)
    y = _tc_mm(agg1, degp, W1, b1, W2)
    agg2 = _agg_kernel(y, src2.reshape(a_shape), dst2.reshape(a_shape),
                       zrows)
    h2 = _tc_h2(agg2, degp, b2)

    u = _pad_sco_idx(jnp.concatenate([pos_edge_index[0], neg_edge_index[0]]))
    v = _pad_sco_idx(jnp.concatenate([pos_edge_index[1], neg_edge_index[1]]))
    part = _score_kernel(h2, u, v)

    sums = _tc_reduce(part.reshape(NW * SCO_PWP, L))
    s = sums.reshape(NW, SCO_PWP)[:, : (2 * P) // NW].reshape(2 * P, 1)
    return (s[:P], s[P:])


# speedup vs baseline: 4.8481x; 1.0168x over previous
import functools

import jax
import jax.numpy as jnp
from jax import lax
from jax.experimental import pallas as pl
from jax.experimental.pallas import tpu as pltpu
from jax.experimental.pallas import tpu_sc as plsc

N = 10000
E = 320000
P = 50000
D_IN = 128
D_HID = 256
D_OUT = 128

NC = 2
NS = 16
NW = NC * NS
L = 16

R = 10240
JUNK = N
TPW = R // NS

AGG_CH = 64
AGG_NCH = 160
AGG_PWP = AGG_NCH * AGG_CH

SCO_CH = 64
SCO_NCH = 52
SCO_PWP = SCO_NCH * SCO_CH
SCO_IR = SCO_NCH // 2

DEG_ROWS = 80

_mesh = plsc.VectorSubcoreMesh(
    core_axis_name="c", subcore_axis_name="s", num_cores=NC, num_subcores=NS)


@functools.partial(
    pl.kernel,
    out_type=jax.ShapeDtypeStruct((NC, 4, R), jnp.float32),
    mesh=_mesh,
    scratch_types=[
        pltpu.VMEM_SHARED((R,), jnp.float32),
        pltpu.VMEM_SHARED((R,), jnp.float32),
        pltpu.VMEM_SHARED((R,), jnp.float32),
        pltpu.VMEM_SHARED((R,), jnp.float32),
        pltpu.VMEM((DEG_ROWS, 128), jnp.int32),
        pltpu.VMEM((DEG_ROWS, 128), jnp.int32),
        pltpu.VMEM((DEG_ROWS, 128), jnp.float32),
        pltpu.SemaphoreType.DMA,
    ],
)
def _deg_kernel(i0_hbm, i1_hbm, i2_hbm, i3_hbm, ones_hbm, zeros_hbm, out_hbm,
                d0, d1, d2, d3, idxa_v, idxb_v, ones_v, semh):
    c = lax.axis_index("c")
    s = lax.axis_index("s")
    wid = s * NC + c
    degs = (d0, d1, d2, d3)
    ins = (i0_hbm, i1_hbm, i2_hbm, i3_hbm)
    for d in degs:
        pltpu.sync_copy(zeros_hbm.at[pl.ds(s * TPW, TPW)],
                        d.at[pl.ds(s * TPW, TPW)])
    plsc.subcore_barrier()
    pltpu.sync_copy(ones_hbm, ones_v)
    pltpu.sync_copy(ins[0].at[wid], idxa_v)
    idxs = (idxa_v, idxb_v)
    for k, d in enumerate(degs):
        buf = idxs[k % 2]

        def body(j, _, d=d, buf=buf):
            pltpu.async_copy(ones_v.at[j], d.at[buf.at[j]], semh, add=True)
            return 0

        lax.fori_loop(0, DEG_ROWS, body, 0)
        if k < 3:
            pltpu.sync_copy(ins[k + 1].at[wid], idxs[(k + 1) % 2])

    def drain(j, _):
        pltpu.make_async_copy(ones_v.at[0], d0.at[idxa_v.at[0]], semh).wait()
        return 0

    lax.fori_loop(0, 4 * DEG_ROWS, drain, 0)
    plsc.subcore_barrier()
    for k, d in enumerate(degs):
        pltpu.sync_copy(d.at[pl.ds(s * TPW, TPW)],
                        out_hbm.at[c, k, pl.ds(s * TPW, TPW)])


@functools.partial(
    pl.kernel,
    out_type=jax.ShapeDtypeStruct((NC, R, 128), jnp.float32),
    mesh=_mesh,
    scratch_types=[
        pltpu.VMEM_SHARED((R, 128), jnp.float32),
        pltpu.VMEM((8, AGG_CH), jnp.int32),
        pltpu.VMEM((8, AGG_CH), jnp.int32),
        pltpu.VMEM((AGG_CH, 128), jnp.float32),
        pltpu.VMEM((AGG_CH, 128), jnp.float32),
        pltpu.VMEM((AGG_CH, 128), jnp.float32),
        pltpu.VMEM((AGG_CH, 128), jnp.float32),
        pltpu.SemaphoreType.DMA,
        pltpu.SemaphoreType.DMA,
        pltpu.SemaphoreType.DMA,
        pltpu.SemaphoreType.DMA,
        pltpu.SemaphoreType.DMA,
        pltpu.SemaphoreType.DMA,
    ],
)
def _agg_kernel(tab_hbm, src_hbm, dst_hbm, zrows_hbm, out_hbm,
                agg_sp, sidx_v, didx_v, r0_v, r1_v, r2_v, r3_v,
                sg0, sg1, sg2, sg3, semd, sems):
    c = lax.axis_index("c")
    s = lax.axis_index("s")
    wid = s * NC + c
    pltpu.sync_copy(zrows_hbm, agg_sp.at[pl.ds(s * TPW, TPW)])
    plsc.subcore_barrier()

    rows = (r0_v, r1_v, r2_v, r3_v)
    gsems = (sg0, sg1, sg2, sg3)

    n_iter = AGG_NCH // 4
    pltpu.async_copy(src_hbm.at[wid, pl.ds(0, 4)],
                     sidx_v.at[pl.ds(0, 4)], semd)
    pltpu.async_copy(dst_hbm.at[wid, pl.ds(0, 4)],
                     didx_v.at[pl.ds(0, 4)], semd)

    def ring(g, _):
        p4 = (g % 2) * 4
        pltpu.make_async_copy(src_hbm.at[wid, pl.ds(0, 4)],
                              sidx_v.at[pl.ds(0, 4)], semd).wait()
        pltpu.make_async_copy(dst_hbm.at[wid, pl.ds(0, 4)],
                              didx_v.at[pl.ds(0, 4)], semd).wait()
        gc = []
        for b in range(4):
            @pl.when(g > 0)
            def _(b=b):
                pltpu.make_async_copy(
                    rows[b], agg_sp.at[didx_v.at[0]], sems).wait()

            gc.append(pltpu.async_copy(
                tab_hbm.at[sidx_v.at[p4 + b]], rows[b], gsems[b]))
        gn = lax.min(g + 1, n_iter - 1)
        pn = ((g + 1) % 2) * 4
        pltpu.async_copy(src_hbm.at[wid, pl.ds(gn * 4, 4)],
                         sidx_v.at[pl.ds(pn, 4)], semd)
        pltpu.async_copy(dst_hbm.at[wid, pl.ds(gn * 4, 4)],
                         didx_v.at[pl.ds(pn, 4)], semd)
        for b in range(4):
            gc[b].wait()
            pltpu.async_copy(rows[b], agg_sp.at[didx_v.at[p4 + b]],
                             sems, add=True)
        return 0

    lax.fori_loop(0, n_iter, ring, 0)
    pltpu.make_async_copy(src_hbm.at[wid, pl.ds(0, 4)],
                          sidx_v.at[pl.ds(0, 4)], semd).wait()
    pltpu.make_async_copy(dst_hbm.at[wid, pl.ds(0, 4)],
                          didx_v.at[pl.ds(0, 4)], semd).wait()
    for b in range(4):
        pltpu.make_async_copy(rows[b], agg_sp.at[didx_v.at[0]], sems).wait()
    plsc.subcore_barrier()
    pltpu.sync_copy(agg_sp.at[pl.ds(s * TPW, TPW)],
                    out_hbm.at[c, pl.ds(s * TPW, TPW)])


@functools.partial(
    pl.kernel,
    out_type=jax.ShapeDtypeStruct((NW, SCO_PWP * L), jnp.float32),
    mesh=_mesh,
    scratch_types=[
        pltpu.VMEM_SHARED((R, 128), jnp.float32),
        pltpu.VMEM((SCO_IR, 128), jnp.int32),
        pltpu.VMEM((SCO_IR, 128), jnp.int32),
        pltpu.VMEM((SCO_CH, 128), jnp.float32),
        pltpu.VMEM((SCO_CH, 128), jnp.float32),
        pltpu.VMEM((SCO_CH, 128), jnp.float32),
        pltpu.VMEM((SCO_CH, 128), jnp.float32),
        pltpu.VMEM((SCO_CH * L,), jnp.float32),
        pltpu.VMEM((SCO_CH * L,), jnp.float32),
        pltpu.SemaphoreType.DMA,
        pltpu.SemaphoreType.DMA,
        pltpu.SemaphoreType.DMA,
        pltpu.SemaphoreType.DMA,
        pltpu.SemaphoreType.DMA,
    ],
)
def _score_kernel(h2_hbm, u_hbm, v_hbm, out_hbm,
                  tab_sp, uidx_v, vidx_v, hu0_v, hv0_v, hu1_v, hv1_v,
                  part0_v, part1_v, su0, sv0, su1, sv1, swb):
    c = lax.axis_index("c")
    s = lax.axis_index("s")
    wid = s * NC + c
    pltpu.sync_copy(h2_hbm.at[pl.ds(s * TPW, TPW)],
                    tab_sp.at[pl.ds(s * TPW, TPW)])
    pltpu.sync_copy(u_hbm.at[wid], uidx_v)
    pltpu.sync_copy(v_hbm.at[wid], vidx_v)
    plsc.subcore_barrier()

    PB = SCO_CH * L

    def compute(hu_v, hv_v, part_v):
        def edge8(t, _):
            for q in range(8):
                e = t * 8 + q
                acc = hu_v[e, pl.ds(0, L)] * hv_v[e, pl.ds(0, L)]
                for k in range(1, 128 // L):
                    acc = acc + (hu_v[e, pl.ds(k * L, L)]
                                 * hv_v[e, pl.ds(k * L, L)])
                part_v[pl.ds(e * L, L)] = acc
            return 0

        lax.fori_loop(0, SCO_CH // 8, edge8, 0)

    def pair(g, _):
        cu0 = pltpu.async_copy(
            tab_sp.at[uidx_v.at[g, pl.ds(0, SCO_CH)]], hu0_v, su0)
        cv0 = pltpu.async_copy(
            tab_sp.at[vidx_v.at[g, pl.ds(0, SCO_CH)]], hv0_v, sv0)
        cu1 = pltpu.async_copy(
            tab_sp.at[uidx_v.at[g, pl.ds(SCO_CH, SCO_CH)]], hu1_v, su1)
        cv1 = pltpu.async_copy(
            tab_sp.at[vidx_v.at[g, pl.ds(SCO_CH, SCO_CH)]], hv1_v, sv1)
        cu0.wait()
        cv0.wait()

        @pl.when(g > 0)
        def _():
            pltpu.make_async_copy(
                part0_v, out_hbm.at[wid, pl.ds(0, PB)], swb).wait()

        compute(hu0_v, hv0_v, part0_v)
        pltpu.async_copy(part0_v, out_hbm.at[wid, pl.ds(g * 2 * PB, PB)], swb)
        cu1.wait()
        cv1.wait()

        @pl.when(g > 0)
        def _():
            pltpu.make_async_copy(
                part1_v, out_hbm.at[wid, pl.ds(0, PB)], swb).wait()

        compute(hu1_v, hv1_v, part1_v)
        pltpu.async_copy(part1_v,
                         out_hbm.at[wid, pl.ds((g * 2 + 1) * PB, PB)], swb)
        return 0

    lax.fori_loop(0, SCO_IR, pair, 0)
    pltpu.make_async_copy(part0_v, out_hbm.at[wid, pl.ds(0, PB)], swb).wait()
    pltpu.make_async_copy(part1_v, out_hbm.at[wid, pl.ds(0, PB)], swb).wait()


def _norm(deg):
    return jnp.where(deg > 0, lax.rsqrt(jnp.maximum(deg, 1e-12)), 0.0)


def _xs_body(x_ref, degp_ref, o_ref):
    deg = degp_ref[0, 0, :] + degp_ref[1, 0, :]
    o_ref[...] = x_ref[...] * _norm(deg)[:, None]


def _tc_xs(x_pad, degp):
    return pl.pallas_call(
        _xs_body,
        out_shape=jax.ShapeDtypeStruct((R, 128), jnp.float32),
    )(x_pad, degp)


_MMB = 1280


def _mm_body(aggp_ref, degp_ref, W1_ref, b1_ref, W2_ref, o_ref):
    p = aggp_ref[0] + aggp_ref[1]
    nd1 = _norm(degp_ref[0, 1, :] + degp_ref[1, 1, :])
    ns2 = _norm(degp_ref[0, 2, :] + degp_ref[1, 2, :])
    h1 = jnp.dot(p * nd1[:, None], W1_ref[...],
                 preferred_element_type=jnp.float32) + b1_ref[...]
    h1 = jnp.maximum(h1, 0.0)
    o_ref[...] = jnp.dot(h1 * ns2[:, None], W2_ref[...],
                         preferred_element_type=jnp.float32)


def _tc_mm(aggp, degp, W1, b1, W2):
    grid = R // _MMB
    return pl.pallas_call(
        _mm_body,
        grid=(grid,),
        in_specs=[
            pl.BlockSpec((NC, _MMB, 128), lambda r: (0, r, 0)),
            pl.BlockSpec((NC, 4, _MMB), lambda r: (0, 0, r)),
            pl.BlockSpec((D_IN, D_HID), lambda r: (0, 0)),
            pl.BlockSpec((1, D_HID), lambda r: (0, 0)),
            pl.BlockSpec((D_HID, D_OUT), lambda r: (0, 0)),
        ],
        out_specs=pl.BlockSpec((_MMB, 128), lambda r: (r, 0)),
        out_shape=jax.ShapeDtypeStruct((R, 128), jnp.float32),
    )(aggp, degp, W1, b1.reshape(1, D_HID), W2)


def _h2_body(aggp_ref, degp_ref, b2_ref, o_ref):
    p = aggp_ref[0] + aggp_ref[1]
    nd2 = _norm(degp_ref[0, 3, :] + degp_ref[1, 3, :])
    o_ref[...] = jnp.maximum(p * nd2[:, None] + b2_ref[...], 0.0)


def _tc_h2(aggp, degp, b2):
    grid = R // _MMB
    return pl.pallas_call(
        _h2_body,
        grid=(grid,),
        in_specs=[
            pl.BlockSpec((NC, _MMB, 128), lambda r: (0, r, 0)),
            pl.BlockSpec((NC, 4, _MMB), lambda r: (0, 0, r)),
            pl.BlockSpec((1, D_OUT), lambda r: (0, 0)),
        ],
        out_specs=pl.BlockSpec((_MMB, 128), lambda r: (r, 0)),
        out_shape=jax.ShapeDtypeStruct((R, 128), jnp.float32),
    )(aggp, degp, b2.reshape(1, D_OUT))


def _red_body(part_ref, o_ref):
    o_ref[...] = jnp.sum(part_ref[...], axis=1, keepdims=True)


def _tc_reduce(part_flat):
    rows = NW * SCO_PWP
    grid = 16
    br = rows // grid
    return pl.pallas_call(
        _red_body,
        grid=(grid,),
        in_specs=[pl.BlockSpec((br, L), lambda r: (r, 0))],
        out_specs=pl.BlockSpec((br, 1), lambda r: (r, 0)),
        out_shape=jax.ShapeDtypeStruct((rows, 1), jnp.float32),
    )(part_flat)


def _pad_edge_idx(idx):
    a = idx.reshape(NW, E // NW)
    pad = jnp.full((NW, AGG_PWP - E // NW), JUNK, jnp.int32)
    return jnp.concatenate([a, pad], axis=1)


def _pad_sco_idx(idx):
    a = idx.reshape(NW, (2 * P) // NW)
    pad = jnp.zeros((NW, SCO_PWP - (2 * P) // NW), jnp.int32)
    return jnp.concatenate([a, pad], axis=1).reshape(NW, SCO_IR, 128)


def kernel(x, block1_edge_index, block2_edge_index, pos_edge_index,
           neg_edge_index, W1, b1, W2, b2):
    ones = jnp.ones((DEG_ROWS, 128), jnp.float32)
    zeros1 = jnp.zeros((R,), jnp.float32)
    zrows = jnp.zeros((TPW, 128), jnp.float32)

    src1 = _pad_edge_idx(block1_edge_index[0])
    dst1 = _pad_edge_idx(block1_edge_index[1])
    src2 = _pad_edge_idx(block2_edge_index[0])
    dst2 = _pad_edge_idx(block2_edge_index[1])

    d_shape = (NW, DEG_ROWS, 128)
    degp = _deg_kernel(src1.reshape(d_shape), dst1.reshape(d_shape),
                       src2.reshape(d_shape), dst2.reshape(d_shape),
                       ones, zeros1)

    x_pad = jnp.concatenate(
        [x, jnp.zeros((R - N, D_IN), jnp.float32)], axis=0)
    xs = _tc_xs(x_pad, degp)

    a_shape = (NW, AGG_NCH, AGG_CH)
    agg1 = _agg_kernel(xs, src1.reshape(a_shape), dst1.reshape(a_shape),
                       zrows)
    y = _tc_mm(agg1, degp, W1, b1, W2)
    agg2 = _agg_kernel(y, src2.reshape(a_shape), dst2.reshape(a_shape),
                       zrows)
    h2 = _tc_h2(agg2, degp, b2)

    u = _pad_sco_idx(jnp.concatenate([pos_edge_index[0], neg_edge_index[0]]))
    v = _pad_sco_idx(jnp.concatenate([pos_edge_index[1], neg_edge_index[1]]))
    part = _score_kernel(h2, u, v)

    sums = _tc_reduce(part.reshape(NW * SCO_PWP, L))
    s = sums.reshape(NW, SCO_PWP)[:, : (2 * P) // NW].reshape(2 * P, 1)
    return (s[:P], s[P:])
